# baseline (speedup 1.0000x reference)
import functools

import jax
import jax.numpy as jnp
from jax import lax
from jax.experimental import pallas as pl
from jax.experimental.pallas import tpu as pltpu
from jax.experimental.pallas import tpu_sc as plsc

N = 10000
E = 320000
D = 128
DQ = D // 4

N_PAD = 10016
E_PAD = 327680
ROWS = E_PAD // 128
ROWS_W = ROWS // 16
EDGES_W = ROWS_W * 128
NROWS_W = N_PAD // 16
BLK = 256
NBLK = EDGES_W // BLK
DBLK = 512


def _tc_proj(x_pad, Wa, a_l2, a_r2):
    blk = 1024

    def body(x_ref, wa_ref, al_ref, ar_ref, zq_ref, zl_ref, zr_ref):
        z = lax.dot_general(x_ref[...], wa_ref[...], (((1,), (1,)), ((), ())),
                            preferred_element_type=jnp.float32)
        for i in range(4):
            zq_ref[i] = z[:, i * DQ:(i + 1) * DQ]
        zl_ref[...] = lax.dot_general(z, al_ref[...], (((1,), (0,)), ((), ())),
                                      preferred_element_type=jnp.float32)
        zr_ref[...] = lax.dot_general(z, ar_ref[...], (((1,), (0,)), ((), ())),
                                      preferred_element_type=jnp.float32)

    return pl.pallas_call(
        body,
        grid=(pl.cdiv(N_PAD, blk),),
        in_specs=[
            pl.BlockSpec((blk, D), lambda i: (i, 0)),
            pl.BlockSpec((D, D), lambda i: (0, 0)),
            pl.BlockSpec((D, 1), lambda i: (0, 0)),
            pl.BlockSpec((D, 1), lambda i: (0, 0)),
        ],
        out_specs=[
            pl.BlockSpec((4, blk, DQ), lambda i: (0, i, 0)),
            pl.BlockSpec((blk, 1), lambda i: (i, 0)),
            pl.BlockSpec((blk, 1), lambda i: (i, 0)),
        ],
        out_shape=[
            jax.ShapeDtypeStruct((4, N_PAD, DQ), jnp.float32),
            jax.ShapeDtypeStruct((N_PAD, 1), jnp.float32),
            jax.ShapeDtypeStruct((N_PAD, 1), jnp.float32),
        ],
    )(x_pad, Wa, a_l2, a_r2)


_mesh = plsc.VectorSubcoreMesh(core_axis_name="c", subcore_axis_name="s")


@functools.partial(
    pl.kernel,
    out_type=jax.ShapeDtypeStruct((4, N_PAD, DQ), jnp.float32),
    mesh=_mesh,
    compiler_params=pltpu.CompilerParams(needs_layout_passes=False,
                                         use_tc_tiling_on_sc=False),
    scratch_types=[
        pltpu.VMEM((EDGES_W,), jnp.int32),
        pltpu.VMEM((EDGES_W,), jnp.int32),
        pltpu.VMEM((EDGES_W,), jnp.float32),
        pltpu.VMEM((N_PAD,), jnp.float32),
        pltpu.VMEM((N_PAD,), jnp.float32),
        pltpu.VMEM((640,), jnp.float32),
        pltpu.VMEM((BLK, DQ), jnp.float32),
        pltpu.VMEM((BLK, DQ), jnp.float32),
        pltpu.VMEM((128, DQ), jnp.float32),
        pltpu.VMEM_SHARED((N_PAD,), jnp.float32),
        pltpu.VMEM_SHARED((N_PAD, DQ), jnp.float32),
        pltpu.SemaphoreType.DMA,
        pltpu.SemaphoreType.DMA,
        pltpu.SemaphoreType.DMA,
        pltpu.SemaphoreType.DMA,
    ],
)
def _sc_edge_kernel(src_hbm, dst_hbm, pw_hbm, zl_hbm, zr_hbm, zq_hbm, out_hbm,
                    src_v, dst_v, pwex_v, zl_v, zr_v, zden_v, rows_a, rows_b,
                    zrows_v, denom_sp, h_sp, sem_a, sem_b, sem_sa, sem_sb):
    c = lax.axis_index("c")
    s = lax.axis_index("s")
    zeros16 = jnp.zeros((16,), jnp.float32)
    nbase = s * NROWS_W

    def zrow(r, carry):
        for k in range(DQ // 16):
            zrows_v[r, pl.ds(k * 16, 16)] = zeros16
        return carry

    lax.fori_loop(0, 128, zrow, 0)
    for k in range(640 // 16):
        zden_v[pl.ds(k * 16, 16)] = zeros16

    @pl.when(s < 15)
    def _():
        pltpu.sync_copy(zden_v.at[pl.ds(0, 632)],
                        denom_sp.at[pl.ds(s * 632, 632)])

    @pl.when(s == 15)
    def _():
        pltpu.sync_copy(zden_v.at[pl.ds(0, N_PAD - 15 * 632)],
                        denom_sp.at[pl.ds(15 * 632, N_PAD - 15 * 632)])

    plsc.subcore_barrier()

    ebase = s * EDGES_W
    pltpu.sync_copy(src_hbm.at[pl.ds(ebase, EDGES_W)], src_v)
    pltpu.sync_copy(dst_hbm.at[pl.ds(ebase, EDGES_W)], dst_v)
    pltpu.sync_copy(pw_hbm.at[pl.ds(ebase, EDGES_W)], pwex_v)
    pltpu.sync_copy(zl_hbm, zl_v)
    pltpu.sync_copy(zr_hbm, zr_v)

    def arow(r, carry):
        for k in range(8):
            sl = pl.ds(r * 128 + k * 16, 16)
            si = src_v[sl]
            di = dst_v[sl]
            pw = pwex_v[sl]
            t = pw * plsc.load_gather(zl_v, [si]) + plsc.load_gather(zr_v, [di])
            e = jnp.maximum(t, 0.01 * t)
            pwex_v[sl] = jnp.exp(e)
        return carry

    lax.fori_loop(0, ROWS_W, arow, 0)

    def dblk(g, carry):
        pltpu.sync_copy(pwex_v.at[pl.ds(g * DBLK, DBLK)],
                        denom_sp.at[dst_v.at[pl.ds(g * DBLK, DBLK)]], add=True)
        return carry

    lax.fori_loop(0, EDGES_W // DBLK, dblk, 0)
    plsc.subcore_barrier()

    pltpu.sync_copy(denom_sp, zl_v)

    def wrow(r, carry):
        for k in range(8):
            sl = pl.ds(r * 128 + k * 16, 16)
            den = plsc.load_gather(zl_v, [dst_v[sl]])
            pwex_v[sl] = pwex_v[sl] / jnp.maximum(den, 1e-16)
        return carry

    lax.fori_loop(0, ROWS_W, wrow, 0)
    plsc.subcore_barrier()

    def fpass(p, pcarry):
        q = 2 * c + p

        for b in range(NROWS_W // 128):
            pltpu.sync_copy(zrows_v, h_sp.at[pl.ds(nbase + b * 128, 128)])
        rem = NROWS_W % 128
        if rem:
            pltpu.sync_copy(zrows_v.at[pl.ds(0, rem)],
                            h_sp.at[pl.ds(nbase + (NROWS_W // 128) * 128,
                                          rem)])
        plsc.subcore_barrier()

        def scale(g, buf):
            def scale_g(gg, c2):
                wv = pwex_v[pl.ds(g * BLK + gg * 16, 16)]
                for jj in range(16):
                    w = wv[jj]
                    row = gg * 16 + jj
                    for k in range(DQ // 16):
                        sl = pl.ds(k * 16, 16)
                        buf[row, sl] = buf[row, sl] * w
                return c2

            lax.fori_loop(0, BLK // 16, scale_g, 0)

        def scatter_start(g, buf, sem):
            def sblk(i, c2):
                pltpu.async_copy(
                    buf.at[pl.ds(i * 128, 128)],
                    h_sp.at[dst_v.at[pl.ds(g * BLK + i * 128, 128)]],
                    sem, add=True)
                return c2

            lax.fori_loop(0, BLK // 128, sblk, 0)

        def scatter_wait(g, buf, sem):
            def swait(i, c2):
                pltpu.make_async_copy(
                    buf.at[pl.ds(i * 128, 128)],
                    h_sp.at[dst_v.at[pl.ds(g * BLK + i * 128, 128)]],
                    sem).wait()
                return c2

            lax.fori_loop(0, BLK // 128, swait, 0)

        def gidx(g):
            return src_v.at[pl.ds(g * BLK, BLK)]

        pltpu.async_copy(zq_hbm.at[q].at[gidx(0)], rows_a, sem_a)
        pltpu.async_copy(zq_hbm.at[q].at[gidx(1)], rows_b, sem_b)

        def hblk2(g2, carry, q=q):
            g = g2 * 2

            @pl.when(g2 > 0)
            def _(g=g):
                scatter_wait(g - 1, rows_b, sem_sb)
                pltpu.async_copy(zq_hbm.at[q].at[gidx(g + 1)], rows_b, sem_b)

            pltpu.make_async_copy(zq_hbm.at[q].at[gidx(g)], rows_a,
                                  sem_a).wait()
            scale(g, rows_a)
            scatter_start(g, rows_a, sem_sa)
            pltpu.make_async_copy(zq_hbm.at[q].at[gidx(g + 1)], rows_b,
                                  sem_b).wait()
            scale(g + 1, rows_b)
            scatter_wait(g, rows_a, sem_sa)

            @pl.when(g + 2 < NBLK)
            def _(g=g):
                pltpu.async_copy(zq_hbm.at[q].at[gidx(g + 2)], rows_a, sem_a)

            scatter_start(g + 1, rows_b, sem_sb)
            return carry

        lax.fori_loop(0, NBLK // 2, hblk2, 0)
        scatter_wait(NBLK - 1, rows_b, sem_sb)
        plsc.subcore_barrier()

        for b in range(NROWS_W // 128):
            sl = pl.ds(nbase + b * 128, 128)
            pltpu.sync_copy(h_sp.at[sl], out_hbm.at[q].at[sl])
        if NROWS_W % 128:
            sl = pl.ds(nbase + (NROWS_W // 128) * 128, NROWS_W % 128)
            pltpu.sync_copy(h_sp.at[sl], out_hbm.at[q].at[sl])
        plsc.subcore_barrier()
        return pcarry

    lax.fori_loop(0, 2, fpass, 0)


def kernel(x, edge_index, pre_w, Wa, a_l, a_r):
    src = edge_index[0]
    dst = edge_index[1]
    pad = E_PAD - E
    pad_dst = N + (jnp.arange(pad, dtype=jnp.int32) % (N_PAD - N))
    src_p = jnp.concatenate([src, jnp.zeros((pad,), jnp.int32)])
    dst_p = jnp.concatenate([dst, pad_dst])
    pw_p = jnp.concatenate([pre_w[:, 0], jnp.zeros((pad,), jnp.float32)])

    x_pad = jnp.pad(x, ((0, N_PAD - N), (0, 0)))
    zq, zl2, zr2 = _tc_proj(x_pad, Wa, a_l.reshape(D, 1), a_r.reshape(D, 1))
    hp = _sc_edge_kernel(src_p, dst_p, pw_p, zl2.reshape(N_PAD),
                         zr2.reshape(N_PAD), zq)
    h = jnp.concatenate([hp[i, :N] for i in range(4)], axis=1)
    return h.reshape(1, N, D)


# speedup vs baseline: 1.0502x; 1.0502x over previous
import functools

import jax
import jax.numpy as jnp
from jax import lax
from jax.experimental import pallas as pl
from jax.experimental.pallas import tpu as pltpu
from jax.experimental.pallas import tpu_sc as plsc

N = 10000
E = 320000
D = 128
DQ = D // 4

N_PAD = 10016
E_PAD = 327680
ROWS = E_PAD // 128
ROWS_W = ROWS // 16
EDGES_W = ROWS_W * 128
NROWS_W = N_PAD // 16
BLK = 256
NBLK = EDGES_W // BLK
DBLK = 512


def _tc_proj(x_pad, Wa, a_l2, a_r2):
    blk = 1024

    def body(x_ref, wa_ref, al_ref, ar_ref, zq_ref, zl_ref, zr_ref):
        z = lax.dot_general(x_ref[...], wa_ref[...], (((1,), (1,)), ((), ())),
                            preferred_element_type=jnp.float32)
        for i in range(4):
            zq_ref[i] = z[:, i * DQ:(i + 1) * DQ]
        zl_ref[...] = lax.dot_general(z, al_ref[...], (((1,), (0,)), ((), ())),
                                      preferred_element_type=jnp.float32)
        zr_ref[...] = lax.dot_general(z, ar_ref[...], (((1,), (0,)), ((), ())),
                                      preferred_element_type=jnp.float32)

    return pl.pallas_call(
        body,
        grid=(pl.cdiv(N_PAD, blk),),
        in_specs=[
            pl.BlockSpec((blk, D), lambda i: (i, 0)),
            pl.BlockSpec((D, D), lambda i: (0, 0)),
            pl.BlockSpec((D, 1), lambda i: (0, 0)),
            pl.BlockSpec((D, 1), lambda i: (0, 0)),
        ],
        out_specs=[
            pl.BlockSpec((4, blk, DQ), lambda i: (0, i, 0)),
            pl.BlockSpec((blk, 1), lambda i: (i, 0)),
            pl.BlockSpec((blk, 1), lambda i: (i, 0)),
        ],
        out_shape=[
            jax.ShapeDtypeStruct((4, N_PAD, DQ), jnp.float32),
            jax.ShapeDtypeStruct((N_PAD, 1), jnp.float32),
            jax.ShapeDtypeStruct((N_PAD, 1), jnp.float32),
        ],
    )(x_pad, Wa, a_l2, a_r2)


_mesh = plsc.VectorSubcoreMesh(core_axis_name="c", subcore_axis_name="s")


@functools.partial(
    pl.kernel,
    out_type=jax.ShapeDtypeStruct((4, N_PAD, DQ), jnp.float32),
    mesh=_mesh,
    compiler_params=pltpu.CompilerParams(needs_layout_passes=False,
                                         use_tc_tiling_on_sc=False),
    scratch_types=[
        pltpu.VMEM((EDGES_W,), jnp.int32),
        pltpu.VMEM((EDGES_W,), jnp.int32),
        pltpu.VMEM((EDGES_W,), jnp.float32),
        pltpu.VMEM((N_PAD,), jnp.float32),
        pltpu.VMEM((N_PAD,), jnp.float32),
        pltpu.VMEM((640,), jnp.float32),
        pltpu.VMEM((BLK, DQ), jnp.float32),
        pltpu.VMEM((BLK, DQ), jnp.float32),
        pltpu.VMEM((128, DQ), jnp.float32),
        pltpu.VMEM_SHARED((N_PAD,), jnp.float32),
        pltpu.VMEM_SHARED((N_PAD, DQ), jnp.float32),
        pltpu.SemaphoreType.DMA,
        pltpu.SemaphoreType.DMA,
        pltpu.SemaphoreType.DMA,
        pltpu.SemaphoreType.DMA,
    ],
)
def _sc_edge_kernel(src_hbm, dst_hbm, pw_hbm, zl_hbm, zr_hbm, zq_hbm, out_hbm,
                    src_v, dst_v, pwex_v, zl_v, zr_v, zden_v, rows_a, rows_b,
                    zrows_v, denom_sp, h_sp, sem_a, sem_b, semd_a, semd_b):
    c = lax.axis_index("c")
    s = lax.axis_index("s")
    zeros16 = jnp.zeros((16,), jnp.float32)
    nbase = s * NROWS_W

    def zrow(r, carry):
        for k in range(DQ // 16):
            zrows_v[r, pl.ds(k * 16, 16)] = zeros16
        return carry

    lax.fori_loop(0, 128, zrow, 0)
    for k in range(640 // 16):
        zden_v[pl.ds(k * 16, 16)] = zeros16

    @pl.when(s < 15)
    def _():
        pltpu.sync_copy(zden_v.at[pl.ds(0, 632)],
                        denom_sp.at[pl.ds(s * 632, 632)])

    @pl.when(s == 15)
    def _():
        pltpu.sync_copy(zden_v.at[pl.ds(0, N_PAD - 15 * 632)],
                        denom_sp.at[pl.ds(15 * 632, N_PAD - 15 * 632)])

    plsc.subcore_barrier()

    ebase = s * EDGES_W
    pltpu.sync_copy(src_hbm.at[pl.ds(ebase, EDGES_W)], src_v)
    pltpu.sync_copy(dst_hbm.at[pl.ds(ebase, EDGES_W)], dst_v)
    pltpu.sync_copy(pw_hbm.at[pl.ds(ebase, EDGES_W)], pwex_v)
    pltpu.sync_copy(zl_hbm, zl_v)
    pltpu.sync_copy(zr_hbm, zr_v)

    def arow(r, carry):
        for k in range(8):
            sl = pl.ds(r * 128 + k * 16, 16)
            si = src_v[sl]
            di = dst_v[sl]
            pw = pwex_v[sl]
            t = pw * plsc.load_gather(zl_v, [si]) + plsc.load_gather(zr_v, [di])
            e = jnp.maximum(t, 0.01 * t)
            pwex_v[sl] = jnp.exp(e)
        return carry

    lax.fori_loop(0, ROWS_W, arow, 0)

    ND = EDGES_W // DBLK

    def dstart(g, sem):
        pltpu.async_copy(pwex_v.at[pl.ds(g * DBLK, DBLK)],
                         denom_sp.at[dst_v.at[pl.ds(g * DBLK, DBLK)]],
                         sem, add=True)

    def dwait(g, sem):
        pltpu.make_async_copy(pwex_v.at[pl.ds(g * DBLK, DBLK)],
                              denom_sp.at[dst_v.at[pl.ds(g * DBLK, DBLK)]],
                              sem).wait()

    dstart(0, semd_a)
    dstart(1, semd_b)

    def dgrp(g2, carry):
        g = g2 * 2
        dwait(g, semd_a)

        @pl.when(g + 2 < ND)
        def _(g=g):
            dstart(g + 2, semd_a)

        dwait(g + 1, semd_b)

        @pl.when(g + 3 < ND)
        def _(g=g):
            dstart(g + 3, semd_b)

        return carry

    lax.fori_loop(0, ND // 2, dgrp, 0)
    plsc.subcore_barrier()

    pltpu.sync_copy(denom_sp, zl_v)

    def wrow(r, carry):
        for k in range(8):
            sl = pl.ds(r * 128 + k * 16, 16)
            den = plsc.load_gather(zl_v, [dst_v[sl]])
            pwex_v[sl] = pwex_v[sl] / jnp.maximum(den, 1e-16)
        return carry

    lax.fori_loop(0, ROWS_W, wrow, 0)
    plsc.subcore_barrier()

    def fpass(p, pcarry):
        q = 2 * c + p

        for b in range(NROWS_W // 128):
            pltpu.sync_copy(zrows_v, h_sp.at[pl.ds(nbase + b * 128, 128)])
        rem = NROWS_W % 128
        if rem:
            pltpu.sync_copy(zrows_v.at[pl.ds(0, rem)],
                            h_sp.at[pl.ds(nbase + (NROWS_W // 128) * 128,
                                          rem)])
        plsc.subcore_barrier()

        def consume(g, buf):
            def scale_g(gg, c2):
                wv = pwex_v[pl.ds(g * BLK + gg * 16, 16)]
                for jj in range(16):
                    w = wv[jj]
                    row = gg * 16 + jj
                    for k in range(DQ // 16):
                        sl = pl.ds(k * 16, 16)
                        buf[row, sl] = buf[row, sl] * w
                return c2

            lax.fori_loop(0, BLK // 16, scale_g, 0)

            def sblk(i, c2):
                pltpu.sync_copy(
                    buf.at[pl.ds(i * 128, 128)],
                    h_sp.at[dst_v.at[pl.ds(g * BLK + i * 128, 128)]],
                    add=True)
                return c2

            lax.fori_loop(0, BLK // 128, sblk, 0)

        def gidx(g):
            return src_v.at[pl.ds(g * BLK, BLK)]

        pltpu.async_copy(zq_hbm.at[q].at[gidx(0)], rows_a, sem_a)

        def hblk2(g2, carry, q=q):
            g = g2 * 2
            pltpu.async_copy(zq_hbm.at[q].at[gidx(g + 1)], rows_b, sem_b)
            pltpu.make_async_copy(zq_hbm.at[q].at[gidx(g)], rows_a,
                                  sem_a).wait()
            consume(g, rows_a)

            @pl.when(g + 2 < NBLK)
            def _(g=g):
                pltpu.async_copy(zq_hbm.at[q].at[gidx(g + 2)], rows_a, sem_a)

            pltpu.make_async_copy(zq_hbm.at[q].at[gidx(g + 1)], rows_b,
                                  sem_b).wait()
            consume(g + 1, rows_b)
            return carry

        lax.fori_loop(0, NBLK // 2, hblk2, 0)
        plsc.subcore_barrier()

        for b in range(NROWS_W // 128):
            sl = pl.ds(nbase + b * 128, 128)
            pltpu.sync_copy(h_sp.at[sl], out_hbm.at[q].at[sl])
        if NROWS_W % 128:
            sl = pl.ds(nbase + (NROWS_W // 128) * 128, NROWS_W % 128)
            pltpu.sync_copy(h_sp.at[sl], out_hbm.at[q].at[sl])
        plsc.subcore_barrier()
        return pcarry

    lax.fori_loop(0, 2, fpass, 0)


def kernel(x, edge_index, pre_w, Wa, a_l, a_r):
    src = edge_index[0]
    dst = edge_index[1]
    pad = E_PAD - E
    pad_dst = N + (jnp.arange(pad, dtype=jnp.int32) % (N_PAD - N))
    src_p = jnp.concatenate([src, jnp.zeros((pad,), jnp.int32)])
    dst_p = jnp.concatenate([dst, pad_dst])
    pw_p = jnp.concatenate([pre_w[:, 0], jnp.zeros((pad,), jnp.float32)])

    x_pad = jnp.pad(x, ((0, N_PAD - N), (0, 0)))
    zq, zl2, zr2 = _tc_proj(x_pad, Wa, a_l.reshape(D, 1), a_r.reshape(D, 1))
    hp = _sc_edge_kernel(src_p, dst_p, pw_p, zl2.reshape(N_PAD),
                         zr2.reshape(N_PAD), zq)
    h = jnp.concatenate([hp[i, :N] for i in range(4)], axis=1)
    return h.reshape(1, N, D)


# speedup vs baseline: 1.0655x; 1.0146x over previous
import functools

import jax
import jax.numpy as jnp
from jax import lax
from jax.experimental import pallas as pl
from jax.experimental.pallas import tpu as pltpu
from jax.experimental.pallas import tpu_sc as plsc

N = 10000
E = 320000
D = 128
DQ = D // 4

N_PAD = 10016
E_PAD = 327680
ROWS = E_PAD // 128
ROWS_W = ROWS // 16
EDGES_W = ROWS_W * 128
NROWS_W = N_PAD // 16
BLK = 256
NBLK = EDGES_W // BLK
DBLK = 512


def _tc_proj(x_pad, Wa, a_l2, a_r2):
    blk = 1024

    def body(x_ref, wa_ref, al_ref, ar_ref, zq_ref, zl_ref, zr_ref):
        z = lax.dot_general(x_ref[...], wa_ref[...], (((1,), (1,)), ((), ())),
                            preferred_element_type=jnp.float32)
        for i in range(4):
            zq_ref[i] = z[:, i * DQ:(i + 1) * DQ]
        zl_ref[...] = lax.dot_general(z, al_ref[...], (((1,), (0,)), ((), ())),
                                      preferred_element_type=jnp.float32)
        zr_ref[...] = lax.dot_general(z, ar_ref[...], (((1,), (0,)), ((), ())),
                                      preferred_element_type=jnp.float32)

    return pl.pallas_call(
        body,
        grid=(pl.cdiv(N_PAD, blk),),
        in_specs=[
            pl.BlockSpec((blk, D), lambda i: (i, 0)),
            pl.BlockSpec((D, D), lambda i: (0, 0)),
            pl.BlockSpec((D, 1), lambda i: (0, 0)),
            pl.BlockSpec((D, 1), lambda i: (0, 0)),
        ],
        out_specs=[
            pl.BlockSpec((4, blk, DQ), lambda i: (0, i, 0)),
            pl.BlockSpec((blk, 1), lambda i: (i, 0)),
            pl.BlockSpec((blk, 1), lambda i: (i, 0)),
        ],
        out_shape=[
            jax.ShapeDtypeStruct((4, N_PAD, DQ), jnp.float32),
            jax.ShapeDtypeStruct((N_PAD, 1), jnp.float32),
            jax.ShapeDtypeStruct((N_PAD, 1), jnp.float32),
        ],
    )(x_pad, Wa, a_l2, a_r2)


_mesh = plsc.VectorSubcoreMesh(core_axis_name="c", subcore_axis_name="s")


@functools.partial(
    pl.kernel,
    out_type=jax.ShapeDtypeStruct((4, N_PAD, DQ), jnp.float32),
    mesh=_mesh,
    compiler_params=pltpu.CompilerParams(needs_layout_passes=False,
                                         use_tc_tiling_on_sc=False),
    scratch_types=[
        pltpu.VMEM((EDGES_W,), jnp.int32),
        pltpu.VMEM((EDGES_W,), jnp.int32),
        pltpu.VMEM((EDGES_W,), jnp.float32),
        pltpu.VMEM((N_PAD,), jnp.float32),
        pltpu.VMEM((N_PAD,), jnp.float32),
        pltpu.VMEM((640,), jnp.float32),
        pltpu.VMEM((BLK, DQ), jnp.float32),
        pltpu.VMEM((BLK, DQ), jnp.float32),
        pltpu.VMEM((128, DQ), jnp.float32),
        pltpu.VMEM_SHARED((N_PAD,), jnp.float32),
        pltpu.VMEM_SHARED((N_PAD, DQ), jnp.float32),
        pltpu.SemaphoreType.DMA,
        pltpu.SemaphoreType.DMA,
        pltpu.SemaphoreType.DMA,
        pltpu.SemaphoreType.DMA,
        pltpu.SemaphoreType.DMA,
    ],
)
def _sc_edge_kernel(src_hbm, dst_hbm, pw_hbm, zl_hbm, zr_hbm, zq_hbm, out_hbm,
                    src_v, dst_v, pwex_v, zl_v, zr_v, zden_v, rows_a, rows_b,
                    zrows_v, denom_sp, h_sp, sem_a, sem_b, semd_a, semd_b,
                    sem_s):
    c = lax.axis_index("c")
    s = lax.axis_index("s")
    zeros16 = jnp.zeros((16,), jnp.float32)
    nbase = s * NROWS_W

    def zrow(r, carry):
        for k in range(DQ // 16):
            zrows_v[r, pl.ds(k * 16, 16)] = zeros16
        return carry

    lax.fori_loop(0, 128, zrow, 0)
    for k in range(640 // 16):
        zden_v[pl.ds(k * 16, 16)] = zeros16

    @pl.when(s < 15)
    def _():
        pltpu.sync_copy(zden_v.at[pl.ds(0, 632)],
                        denom_sp.at[pl.ds(s * 632, 632)])

    @pl.when(s == 15)
    def _():
        pltpu.sync_copy(zden_v.at[pl.ds(0, N_PAD - 15 * 632)],
                        denom_sp.at[pl.ds(15 * 632, N_PAD - 15 * 632)])

    plsc.subcore_barrier()

    ebase = s * EDGES_W
    pltpu.sync_copy(src_hbm.at[pl.ds(ebase, EDGES_W)], src_v)
    pltpu.sync_copy(dst_hbm.at[pl.ds(ebase, EDGES_W)], dst_v)
    pltpu.sync_copy(pw_hbm.at[pl.ds(ebase, EDGES_W)], pwex_v)
    pltpu.sync_copy(zl_hbm, zl_v)
    pltpu.sync_copy(zr_hbm, zr_v)

    def arow(r, carry):
        for k in range(8):
            sl = pl.ds(r * 128 + k * 16, 16)
            si = src_v[sl]
            di = dst_v[sl]
            pw = pwex_v[sl]
            t = pw * plsc.load_gather(zl_v, [si]) + plsc.load_gather(zr_v, [di])
            e = jnp.maximum(t, 0.01 * t)
            pwex_v[sl] = jnp.exp(e)
        return carry

    lax.fori_loop(0, ROWS_W, arow, 0)

    ND = EDGES_W // DBLK

    def dstart(g, sem):
        pltpu.async_copy(pwex_v.at[pl.ds(g * DBLK, DBLK)],
                         denom_sp.at[dst_v.at[pl.ds(g * DBLK, DBLK)]],
                         sem, add=True)

    def dwait(g, sem):
        pltpu.make_async_copy(pwex_v.at[pl.ds(g * DBLK, DBLK)],
                              denom_sp.at[dst_v.at[pl.ds(g * DBLK, DBLK)]],
                              sem).wait()

    dstart(0, semd_a)
    dstart(1, semd_b)

    def dgrp(g2, carry):
        g = g2 * 2
        dwait(g, semd_a)

        @pl.when(g + 2 < ND)
        def _(g=g):
            dstart(g + 2, semd_a)

        dwait(g + 1, semd_b)

        @pl.when(g + 3 < ND)
        def _(g=g):
            dstart(g + 3, semd_b)

        return carry

    lax.fori_loop(0, ND // 2, dgrp, 0)
    plsc.subcore_barrier()

    pltpu.sync_copy(denom_sp, zl_v)

    def wrow(r, carry):
        for k in range(8):
            sl = pl.ds(r * 128 + k * 16, 16)
            den = plsc.load_gather(zl_v, [dst_v[sl]])
            pwex_v[sl] = pwex_v[sl] / jnp.maximum(den, 1e-16)
        return carry

    lax.fori_loop(0, ROWS_W, wrow, 0)
    plsc.subcore_barrier()

    def fpass(p, pcarry):
        q = 2 * c + p

        for b in range(NROWS_W // 128):
            pltpu.sync_copy(zrows_v, h_sp.at[pl.ds(nbase + b * 128, 128)])
        rem = NROWS_W % 128
        if rem:
            pltpu.sync_copy(zrows_v.at[pl.ds(0, rem)],
                            h_sp.at[pl.ds(nbase + (NROWS_W // 128) * 128,
                                          rem)])
        plsc.subcore_barrier()

        def consume(g, buf):
            def scale_half(h0):
                def scale_g(gg, c2):
                    wv = pwex_v[pl.ds(g * BLK + h0 + gg * 16, 16)]
                    for jj in range(16):
                        w = wv[jj]
                        row = h0 + gg * 16 + jj
                        for k in range(DQ // 16):
                            sl = pl.ds(k * 16, 16)
                            buf[row, sl] = buf[row, sl] * w
                    return c2

                lax.fori_loop(0, BLK // 32, scale_g, 0)

            scale_half(0)
            pltpu.async_copy(buf.at[pl.ds(0, 128)],
                             h_sp.at[dst_v.at[pl.ds(g * BLK, 128)]],
                             sem_s, add=True)
            scale_half(BLK // 2)
            pltpu.sync_copy(buf.at[pl.ds(128, 128)],
                            h_sp.at[dst_v.at[pl.ds(g * BLK + 128, 128)]],
                            add=True)
            pltpu.make_async_copy(buf.at[pl.ds(0, 128)],
                                  h_sp.at[dst_v.at[pl.ds(g * BLK, 128)]],
                                  sem_s).wait()

        def gidx(g):
            return src_v.at[pl.ds(g * BLK, BLK)]

        pltpu.async_copy(zq_hbm.at[q].at[gidx(0)], rows_a, sem_a)

        def hblk2(g2, carry, q=q):
            g = g2 * 2
            pltpu.async_copy(zq_hbm.at[q].at[gidx(g + 1)], rows_b, sem_b)
            pltpu.make_async_copy(zq_hbm.at[q].at[gidx(g)], rows_a,
                                  sem_a).wait()
            consume(g, rows_a)

            @pl.when(g + 2 < NBLK)
            def _(g=g):
                pltpu.async_copy(zq_hbm.at[q].at[gidx(g + 2)], rows_a, sem_a)

            pltpu.make_async_copy(zq_hbm.at[q].at[gidx(g + 1)], rows_b,
                                  sem_b).wait()
            consume(g + 1, rows_b)
            return carry

        lax.fori_loop(0, NBLK // 2, hblk2, 0)
        plsc.subcore_barrier()

        for b in range(NROWS_W // 128):
            sl = pl.ds(nbase + b * 128, 128)
            pltpu.sync_copy(h_sp.at[sl], out_hbm.at[q].at[sl])
        if NROWS_W % 128:
            sl = pl.ds(nbase + (NROWS_W // 128) * 128, NROWS_W % 128)
            pltpu.sync_copy(h_sp.at[sl], out_hbm.at[q].at[sl])
        plsc.subcore_barrier()
        return pcarry

    lax.fori_loop(0, 2, fpass, 0)


def kernel(x, edge_index, pre_w, Wa, a_l, a_r):
    src = edge_index[0]
    dst = edge_index[1]
    pad = E_PAD - E
    pad_dst = N + (jnp.arange(pad, dtype=jnp.int32) % (N_PAD - N))
    src_p = jnp.concatenate([src, jnp.zeros((pad,), jnp.int32)])
    dst_p = jnp.concatenate([dst, pad_dst])
    pw_p = jnp.concatenate([pre_w[:, 0], jnp.zeros((pad,), jnp.float32)])

    x_pad = jnp.pad(x, ((0, N_PAD - N), (0, 0)))
    zq, zl2, zr2 = _tc_proj(x_pad, Wa, a_l.reshape(D, 1), a_r.reshape(D, 1))
    hp = _sc_edge_kernel(src_p, dst_p, pw_p, zl2.reshape(N_PAD),
                         zr2.reshape(N_PAD), zq)
    h = jnp.concatenate([hp[i, :N] for i in range(4)], axis=1)
    return h.reshape(1, N, D)
